# Initial kernel scaffold; baseline (speedup 1.0000x reference)
import jax
import jax.numpy as jnp
from jax.experimental import pallas as pl


def kernel(x, edge_index, batch, conv_w, conv_b, dec_w, dec_b, W1, b1, W2, b2, lin_w, lin_b):
    raise NotImplementedError("write your pallas kernel here")


# speedup vs baseline: 2.3987x; 2.3987x over previous
import functools

import numpy as np
import jax
import jax.numpy as jnp
from jax import lax
from jax.experimental import pallas as pl
from jax.experimental.pallas import tpu as pltpu
import jax.experimental.pallas.tpu_sc as plsc

SIZE = 256
B = 8
NPG = 896
E = 229376
HID_ENC = 16
HID_GCN = 64
NUM_CLASSES = 10
CONV_HID = 32
N = B * NPG


def _create_circular(h, w, center=None, tolerance=1, min_val=0):
    if center is None:
        center = (w / 2 - 0.5, h / 2 - 0.5)
    dist = np.zeros((h, w))
    for i in range(h):
        for j in range(w):
            dist[i][j] = int(max(abs(i - center[0]), abs(j - center[1])) // tolerance + min_val)
    return dist

def _split_tensor(size, split_coefs=None):
    if split_coefs is None:
        split_coefs = [1, 2, 4]
    result = []
    last_val = 0
    for coef in split_coefs:
        step = size // coef
        row_data = []
        for y in range(0, size, step):
            step_result = []
            for x in range(0, size, step):
                mask = _create_circular(step, step, min_val=last_val)
                last_val = mask[-1][-1] + 1
                step_result.append(mask)
            row_data.append(np.concatenate(step_result, axis=1))
        result.append(np.concatenate(row_data, axis=0))
    return result

def _create_border_leng(img_shape, split_coefs=None):
    border_leng = []
    if split_coefs is None:
        split_coefs = [1, 2, 4]
    for coef in split_coefs:
        for _ in range(coef ** 2):
            inp = 2
            for _ in range(img_shape // (2 * coef)):
                border_leng.append(inp)
                inp += 2
    return border_leng

def _reshape_2d(array, border_size):
    ptr_st, ptr_end = border_size, len(array) - border_size
    data1 = list(array[:ptr_st])
    data2 = list(array[ptr_end:])
    itt = 4
    while ptr_st != ptr_end:
        if itt > 2:
            data1.append(array[ptr_st])
        else:
            data2.append(array[ptr_st])
        ptr_st += 1
        itt -= 1
        if itt == 0:
            itt = 4
    return np.stack([np.asarray(data1), np.asarray(data2)])

def _build_static():
    gmap = _split_tensor(SIZE)
    border = _create_border_leng(SIZE)
    gather = []
    minv = 0
    for m, vm in enumerate(gmap):
        vmi = vm.astype(int)
        maxv = int(vmi[-1][-1])
        flat = vmi.reshape(-1)
        for rad in range(minv, maxv + 1):
            pos = np.nonzero(flat == rad)[0] + m * SIZE * SIZE
            gather.append(_reshape_2d(pos, border[rad]).astype(np.int64))
        minv = maxv + 1
    assert len(gather) == NPG

    CH = 128
    Ls = [g.shape[1] for g in gather]
    P = [((L - 1 + CH - 1) // CH) * CH for L in Ls]
    Tpack = int(np.sum(P))
    M = Tpack // CH

    tap = np.zeros((4, Tpack), dtype=np.int64)
    di = np.zeros((CONV_HID, Tpack), dtype=np.int64)
    rid = np.zeros((M,), dtype=np.int32)
    first = np.zeros((M,), dtype=np.int32)
    ZCOL = int(np.sum([CONV_HID * (L - 1) for L in Ls]))
    di[:] = ZCOL
    toff = 0
    foff = 0
    moff = 0
    for r in range(NPG):
        G = gather[r]
        L = Ls[r]
        T = L - 1
        tap[0, toff:toff + T] = G[0, :T]
        tap[1, toff:toff + T] = G[0, 1:L]
        tap[2, toff:toff + T] = G[1, :T]
        tap[3, toff:toff + T] = G[1, 1:L]
        for c in range(CONV_HID):
            di[c, toff:toff + T] = foff + c * T + np.arange(T)
        nch = P[r] // CH
        rid[moff:moff + nch] = r
        first[moff] = 1
        toff += P[r]
        foff += CONV_HID * T
        moff += nch
    return tap.astype(np.int32), di.astype(np.int32), rid, first, Tpack, M

_TAP_NP, _DI_NP, _RID_NP, _FIRST_NP, _TPACK, _M = _build_static()
_TAP = jnp.asarray(_TAP_NP)
_DI = jnp.asarray(_DI_NP)
_RID = jnp.asarray(_RID_NP)
_FIRST = jnp.asarray(_FIRST_NP)


def _enc_body(rid_ref, fst_ref, xg_ref, w_ref, b_ref, dec_ref, db_ref, out_ref):
    m = pl.program_id(0)
    taps = xg_ref[...]
    w = w_ref[...]
    bias = b_ref[...]
    a = (taps[:, 0:1, :] * w[0, :, 0][None, :, None]
         + taps[:, 1:2, :] * w[0, :, 1][None, :, None]
         + taps[:, 2:3, :] * w[0, :, 2][None, :, None]
         + taps[:, 3:4, :] * w[0, :, 3][None, :, None])
    a = a + bias[0, 0, :][None, :, None]
    a = jnp.maximum(a, 0.0)
    a2 = a.reshape(B, CONV_HID * 128)
    d2 = dec_ref[...].reshape(HID_ENC, CONV_HID * 128)
    res = lax.dot_general(a2, d2, (((1,), (1,)), ((), ())),
                          preferred_element_type=jnp.float32)
    fst = fst_ref[m]

    @pl.when(fst == 1)
    def _():
        out_ref[...] = res[None] + db_ref[...]

    @pl.when(fst == 0)
    def _():
        out_ref[...] = out_ref[...] + res[None]


def _encoder(xg, convw, convb, dec_packed, decb):
    grid_spec = pltpu.PrefetchScalarGridSpec(
        num_scalar_prefetch=2,
        grid=(_M,),
        in_specs=[
            pl.BlockSpec((B, 4, 128), lambda m, rid, fst: (0, 0, m)),
            pl.BlockSpec((1, CONV_HID, 4), lambda m, rid, fst: (rid[m], 0, 0)),
            pl.BlockSpec((1, 1, CONV_HID), lambda m, rid, fst: (rid[m], 0, 0)),
            pl.BlockSpec((HID_ENC, CONV_HID, 128), lambda m, rid, fst: (0, 0, m)),
            pl.BlockSpec((1, 1, HID_ENC), lambda m, rid, fst: (rid[m], 0, 0)),
        ],
        out_specs=pl.BlockSpec((1, B, HID_ENC), lambda m, rid, fst: (rid[m], 0, 0)),
    )
    return pl.pallas_call(
        _enc_body,
        grid_spec=grid_spec,
        out_shape=jax.ShapeDtypeStruct((NPG, B, HID_ENC), jnp.float32),
    )(_RID, _FIRST, xg, convw, convb, dec_packed, decb)


_NC, _NS = 2, 16
_NW = _NC * _NS
_EPW = E // _NW
_KE = 128
_NCH = _EPW // _KE
_RPT = N // _NS

_DW = 16


@functools.cache
def _deg_kernel_fn():
    mesh = plsc.VectorSubcoreMesh(core_axis_name="c", subcore_axis_name="s")
    return functools.partial(
        pl.kernel,
        out_type=jax.ShapeDtypeStruct((_NC, N, _DW), jnp.float32),
        mesh=mesh,
        compiler_params=pltpu.CompilerParams(use_tc_tiling_on_sc=False),
        scratch_types=[
            pltpu.VMEM((_KE,), jnp.int32),
            pltpu.VMEM((_KE, _DW), jnp.float32),
            pltpu.VMEM_SHARED((N, _DW), jnp.float32),
        ],
    )(_deg_body)


def _deg_body(dst_hbm, ones_hbm, zerosS_hbm, out_hbm, dst_v, ones_v, acc):
    cid = lax.axis_index("c")
    sid = lax.axis_index("s")
    w = cid * _NS + sid
    pltpu.sync_copy(ones_hbm, ones_v)
    pltpu.sync_copy(zerosS_hbm.at[pl.ds(sid * _RPT, _RPT)],
                    acc.at[pl.ds(sid * _RPT, _RPT)])
    plsc.subcore_barrier()

    def body(j, carry):
        base = w * _EPW + j * _KE
        pltpu.sync_copy(dst_hbm.at[pl.ds(base, _KE)], dst_v)
        pltpu.sync_copy(ones_v, acc.at[dst_v], add=True)
        return carry

    lax.fori_loop(0, _NCH, body, 0)
    plsc.subcore_barrier()
    pltpu.sync_copy(acc.at[pl.ds(sid * _RPT, _RPT)],
                    out_hbm.at[cid, pl.ds(sid * _RPT, _RPT)])


@functools.cache
def _edge_kernel_fn():
    mesh = plsc.VectorSubcoreMesh(core_axis_name="c", subcore_axis_name="s")
    return functools.partial(
        pl.kernel,
        out_type=jax.ShapeDtypeStruct((_NC, N, HID_GCN), jnp.float32),
        mesh=mesh,
        compiler_params=pltpu.CompilerParams(use_tc_tiling_on_sc=False),
        scratch_types=[
            pltpu.VMEM((_KE,), jnp.int32),
            pltpu.VMEM((_KE,), jnp.int32),
            pltpu.VMEM((_KE, HID_GCN), jnp.float32),
            pltpu.VMEM_SHARED((N, HID_GCN), jnp.float32),
            pltpu.SemaphoreType.DMA,
        ],
    )(_edge_body)


def _edge_body(g_hbm, src_hbm, dst_hbm, zeros_hbm, out_hbm,
               src_v, dst_v, rows_v, acc, sem):
    cid = lax.axis_index("c")
    sid = lax.axis_index("s")
    w = cid * _NS + sid
    pltpu.sync_copy(zeros_hbm.at[pl.ds(sid * _RPT, _RPT)],
                    acc.at[pl.ds(sid * _RPT, _RPT)])
    plsc.subcore_barrier()

    def body(j, carry):
        base = w * _EPW + j * _KE
        pltpu.sync_copy(src_hbm.at[pl.ds(base, _KE)], src_v)
        pltpu.sync_copy(dst_hbm.at[pl.ds(base, _KE)], dst_v)
        pltpu.async_copy(g_hbm.at[src_v], rows_v, sem).wait()
        pltpu.sync_copy(rows_v, acc.at[dst_v], add=True)
        return carry

    lax.fori_loop(0, _NCH, body, 0)
    plsc.subcore_barrier()
    pltpu.sync_copy(acc.at[pl.ds(sid * _RPT, _RPT)],
                    out_hbm.at[cid, pl.ds(sid * _RPT, _RPT)])


def _bk_body(h_ref, w1_ref, degpT_ref, b1_ref, g_ref, sl_ref, dv_ref):
    h = h_ref[...]
    hp = lax.dot_general(h, w1_ref[...], (((1,), (1,)), ((), ())),
                         preferred_element_type=jnp.float32)
    dp = degpT_ref[...]
    deg = dp[:, 0:1] + dp[:, 1:2] + 1.0
    dinv = lax.rsqrt(deg)
    g_ref[...] = dinv * hp
    sl_ref[...] = dinv * dinv * hp + b1_ref[...]
    dv_ref[...] = jnp.broadcast_to(dinv, (N, HID_GCN))


def _ck_body(p_ref, sl_ref, dv_ref, w2_ref, b2_ref, g2_ref, sl2_ref):
    p = p_ref[...]
    dv = dv_ref[...]
    h1 = jnp.maximum(dv * (p[0] + p[1]) + sl_ref[...], 0.0)
    hp = lax.dot_general(h1, w2_ref[...], (((1,), (1,)), ((), ())),
                         preferred_element_type=jnp.float32)
    g2_ref[...] = dv * hp
    sl2_ref[...] = dv * dv * hp + b2_ref[...]


def _dk_body(p_ref, sl2_ref, dv_ref, batch_ref, lw_ref, lb_ref, out_ref):
    p = p_ref[...]
    h2 = dv_ref[...] * (p[0] + p[1]) + sl2_ref[...]
    bc = batch_ref[...]
    ids = lax.broadcasted_iota(jnp.int32, (1, B), 1)
    oneh = (bc == ids).astype(jnp.float32)
    sums = lax.dot_general(oneh, h2, (((0,), (0,)), ((), ())),
                           preferred_element_type=jnp.float32)
    cnts = lax.dot_general(oneh, jnp.ones((N, 1), jnp.float32),
                           (((0,), (0,)), ((), ())),
                           preferred_element_type=jnp.float32)
    pooled = sums / jnp.maximum(cnts, 1.0)
    logits = lax.dot_general(pooled, lw_ref[...], (((1,), (1,)), ((), ())),
                             preferred_element_type=jnp.float32) + lb_ref[...]
    z = logits - jnp.max(logits, axis=1, keepdims=True)
    e = jnp.exp(z)
    out_ref[...] = e / jnp.sum(e, axis=1, keepdims=True)


def _tc_call(body, out_shapes, *args):
    return pl.pallas_call(body, out_shape=out_shapes)(*args)


def kernel(x, edge_index, batch, conv_w, conv_b, dec_w, dec_b,
           W1, b1, W2, b2, lin_w, lin_b):
    f32 = jnp.float32
    xf = x.reshape(B, -1)
    xg = jnp.take(xf, _TAP, axis=1)
    convw = conv_w.reshape(NPG, CONV_HID, 4)
    convb = conv_b.reshape(NPG, 1, CONV_HID)
    dec_flat = jnp.concatenate([w for w in dec_w], axis=1)
    dec_flat = jnp.pad(dec_flat, ((0, 0), (0, 1)))
    dec_packed = jnp.take(dec_flat, _DI, axis=1)
    decb = jnp.stack(dec_b).reshape(NPG, 1, HID_ENC)

    enc = _encoder(xg, convw, convb, dec_packed, decb)
    h = enc.transpose(1, 0, 2).reshape(N, HID_ENC)

    ei = edge_index.astype(jnp.int32)
    src, dst = ei[0], ei[1]
    onesS = jnp.ones((_KE, _DW), f32)
    zerosS = jnp.zeros((N, _DW), f32)
    zerosNK = jnp.zeros((N, HID_GCN), f32)

    degp = _deg_kernel_fn()(dst, onesS, zerosS)
    degpT = degp[:, :, 0].T

    g1, sl1, dv = _tc_call(
        _bk_body,
        [jax.ShapeDtypeStruct((N, HID_GCN), f32)] * 3,
        h, W1, degpT, b1.reshape(1, HID_GCN))

    p1 = _edge_kernel_fn()(g1, src, dst, zerosNK)

    g2, sl2 = _tc_call(
        _ck_body,
        [jax.ShapeDtypeStruct((N, HID_GCN), f32)] * 2,
        p1, sl1, dv, W2, b2.reshape(1, HID_GCN))

    p2 = _edge_kernel_fn()(g2, src, dst, zerosNK)

    out = _tc_call(
        _dk_body,
        jax.ShapeDtypeStruct((B, NUM_CLASSES), f32),
        p2, sl2, dv, batch.astype(jnp.int32).reshape(N, 1),
        lin_w, lin_b.reshape(1, NUM_CLASSES))
    return out


# speedup vs baseline: 10.0333x; 4.1827x over previous
import functools

import numpy as np
import jax
import jax.numpy as jnp
from jax import lax
from jax.experimental import pallas as pl
from jax.experimental.pallas import tpu as pltpu
import jax.experimental.pallas.tpu_sc as plsc

SIZE = 256
B = 8
NPG = 896
E = 229376
HID_ENC = 16
HID_GCN = 64
NUM_CLASSES = 10
CONV_HID = 32
N = B * NPG


def _create_circular(h, w, center=None, tolerance=1, min_val=0):
    if center is None:
        center = (w / 2 - 0.5, h / 2 - 0.5)
    dist = np.zeros((h, w))
    for i in range(h):
        for j in range(w):
            dist[i][j] = int(max(abs(i - center[0]), abs(j - center[1])) // tolerance + min_val)
    return dist

def _split_tensor(size, split_coefs=None):
    if split_coefs is None:
        split_coefs = [1, 2, 4]
    result = []
    last_val = 0
    for coef in split_coefs:
        step = size // coef
        row_data = []
        for y in range(0, size, step):
            step_result = []
            for x in range(0, size, step):
                mask = _create_circular(step, step, min_val=last_val)
                last_val = mask[-1][-1] + 1
                step_result.append(mask)
            row_data.append(np.concatenate(step_result, axis=1))
        result.append(np.concatenate(row_data, axis=0))
    return result

def _create_border_leng(img_shape, split_coefs=None):
    border_leng = []
    if split_coefs is None:
        split_coefs = [1, 2, 4]
    for coef in split_coefs:
        for _ in range(coef ** 2):
            inp = 2
            for _ in range(img_shape // (2 * coef)):
                border_leng.append(inp)
                inp += 2
    return border_leng

def _reshape_2d(array, border_size):
    ptr_st, ptr_end = border_size, len(array) - border_size
    data1 = list(array[:ptr_st])
    data2 = list(array[ptr_end:])
    itt = 4
    while ptr_st != ptr_end:
        if itt > 2:
            data1.append(array[ptr_st])
        else:
            data2.append(array[ptr_st])
        ptr_st += 1
        itt -= 1
        if itt == 0:
            itt = 4
    return np.stack([np.asarray(data1), np.asarray(data2)])

def _build_static():
    gmap = _split_tensor(SIZE)
    border = _create_border_leng(SIZE)
    gather = []
    minv = 0
    for m, vm in enumerate(gmap):
        vmi = vm.astype(int)
        maxv = int(vmi[-1][-1])
        flat = vmi.reshape(-1)
        for rad in range(minv, maxv + 1):
            pos = np.nonzero(flat == rad)[0] + m * SIZE * SIZE
            gather.append(_reshape_2d(pos, border[rad]).astype(np.int64))
        minv = maxv + 1
    assert len(gather) == NPG

    CH = 128
    Ls = [g.shape[1] for g in gather]
    P = [((L - 1 + CH - 1) // CH) * CH for L in Ls]
    Tpack = int(np.sum(P))
    M = Tpack // CH

    tap = np.zeros((4, Tpack), dtype=np.int64)
    di = np.zeros((CONV_HID, Tpack), dtype=np.int64)
    rid = np.zeros((M,), dtype=np.int32)
    first = np.zeros((M,), dtype=np.int32)
    ZCOL = int(np.sum([CONV_HID * (L - 1) for L in Ls]))
    di[:] = ZCOL
    toff = 0
    foff = 0
    moff = 0
    for r in range(NPG):
        G = gather[r]
        L = Ls[r]
        T = L - 1
        tap[0, toff:toff + T] = G[0, :T]
        tap[1, toff:toff + T] = G[0, 1:L]
        tap[2, toff:toff + T] = G[1, :T]
        tap[3, toff:toff + T] = G[1, 1:L]
        for c in range(CONV_HID):
            di[c, toff:toff + T] = foff + c * T + np.arange(T)
        nch = P[r] // CH
        rid[moff:moff + nch] = r
        first[moff] = 1
        toff += P[r]
        foff += CONV_HID * T
        moff += nch
    return tap.astype(np.int32), di.astype(np.int32), rid, first, Tpack, M, Ls, P

_TAP_NP, _DI_NP, _RID_NP, _FIRST_NP, _TPACK, _M, _LS, _PS = _build_static()
_TAP = jnp.asarray(_TAP_NP)
_TAPF = jnp.asarray(_TAP_NP.reshape(-1))
_RID = jnp.asarray(_RID_NP)
_FIRST = jnp.asarray(_FIRST_NP)


def _enc_body(rid_ref, fst_ref, xg_ref, w_ref, b_ref, dec_ref, db_ref, out_ref):
    m = pl.program_id(0)
    taps = xg_ref[...]
    w = w_ref[...]
    bias = b_ref[...]
    a = (taps[:, 0:1, :] * w[0, :, 0][None, :, None]
         + taps[:, 1:2, :] * w[0, :, 1][None, :, None]
         + taps[:, 2:3, :] * w[0, :, 2][None, :, None]
         + taps[:, 3:4, :] * w[0, :, 3][None, :, None])
    a = a + bias[0, 0, :][None, :, None]
    a = jnp.maximum(a, 0.0)
    a2 = a.reshape(B, CONV_HID * 128)
    d2 = dec_ref[...].reshape(HID_ENC, CONV_HID * 128)
    res = lax.dot_general(a2, d2, (((1,), (1,)), ((), ())),
                          preferred_element_type=jnp.float32)
    fst = fst_ref[m]

    @pl.when(fst == 1)
    def _():
        out_ref[...] = res[None] + db_ref[...]

    @pl.when(fst == 0)
    def _():
        out_ref[...] = out_ref[...] + res[None]


def _encoder(xg, convw, convb, dec_packed, decb):
    grid_spec = pltpu.PrefetchScalarGridSpec(
        num_scalar_prefetch=2,
        grid=(_M,),
        in_specs=[
            pl.BlockSpec((B, 4, 128), lambda m, rid, fst: (0, 0, m)),
            pl.BlockSpec((1, CONV_HID, 4), lambda m, rid, fst: (rid[m], 0, 0)),
            pl.BlockSpec((1, 1, CONV_HID), lambda m, rid, fst: (rid[m], 0, 0)),
            pl.BlockSpec((HID_ENC, CONV_HID, 128), lambda m, rid, fst: (0, 0, m)),
            pl.BlockSpec((1, 1, HID_ENC), lambda m, rid, fst: (rid[m], 0, 0)),
        ],
        out_specs=pl.BlockSpec((1, B, HID_ENC), lambda m, rid, fst: (rid[m], 0, 0)),
    )
    return pl.pallas_call(
        _enc_body,
        grid_spec=grid_spec,
        out_shape=jax.ShapeDtypeStruct((NPG, B, HID_ENC), jnp.float32),
    )(_RID, _FIRST, xg, convw, convb, dec_packed, decb)


_NC, _NS = 2, 16
_NW = _NC * _NS
_EPW = E // _NW
_KE = 128
_NCH = _EPW // _KE
_RPT = N // _NS

_DW = 16


@functools.cache
def _deg_kernel_fn():
    mesh = plsc.VectorSubcoreMesh(core_axis_name="c", subcore_axis_name="s")
    return functools.partial(
        pl.kernel,
        out_type=jax.ShapeDtypeStruct((_NC, N, _DW), jnp.float32),
        mesh=mesh,
        compiler_params=pltpu.CompilerParams(use_tc_tiling_on_sc=False),
        scratch_types=[
            pltpu.VMEM((_KE,), jnp.int32),
            pltpu.VMEM((_KE, _DW), jnp.float32),
            pltpu.VMEM_SHARED((N, _DW), jnp.float32),
        ],
    )(_deg_body)


def _deg_body(dst_hbm, ones_hbm, zerosS_hbm, out_hbm, dst_v, ones_v, acc):
    cid = lax.axis_index("c")
    sid = lax.axis_index("s")
    w = cid * _NS + sid
    pltpu.sync_copy(ones_hbm, ones_v)
    pltpu.sync_copy(zerosS_hbm.at[pl.ds(sid * _RPT, _RPT)],
                    acc.at[pl.ds(sid * _RPT, _RPT)])
    plsc.subcore_barrier()

    def body(j, carry):
        base = w * _EPW + j * _KE
        pltpu.sync_copy(dst_hbm.at[pl.ds(base, _KE)], dst_v)
        pltpu.sync_copy(ones_v, acc.at[dst_v], add=True)
        return carry

    lax.fori_loop(0, _NCH, body, 0)
    plsc.subcore_barrier()
    pltpu.sync_copy(acc.at[pl.ds(sid * _RPT, _RPT)],
                    out_hbm.at[cid, pl.ds(sid * _RPT, _RPT)])


@functools.cache
def _edge_kernel_fn():
    mesh = plsc.VectorSubcoreMesh(core_axis_name="c", subcore_axis_name="s")
    return functools.partial(
        pl.kernel,
        out_type=jax.ShapeDtypeStruct((_NC, N, HID_GCN), jnp.float32),
        mesh=mesh,
        compiler_params=pltpu.CompilerParams(use_tc_tiling_on_sc=False),
        scratch_types=[
            pltpu.VMEM((_KE,), jnp.int32),
            pltpu.VMEM((_KE,), jnp.int32),
            pltpu.VMEM((_KE, HID_GCN), jnp.float32),
            pltpu.VMEM_SHARED((N, HID_GCN), jnp.float32),
            pltpu.SemaphoreType.DMA,
        ],
    )(_edge_body)


_NTAP = 4 * _TPACK
_TPW = _NTAP // _NW
_KT = 128
_TNCH = _TPW // _KT


@functools.cache
def _tapgather_kernel_fn():
    mesh = plsc.VectorSubcoreMesh(core_axis_name="c", subcore_axis_name="s")
    return functools.partial(
        pl.kernel,
        out_type=jax.ShapeDtypeStruct((_NTAP, 16), jnp.float32),
        mesh=mesh,
        compiler_params=pltpu.CompilerParams(use_tc_tiling_on_sc=False),
        scratch_types=[
            pltpu.VMEM((_KT,), jnp.int32),
            pltpu.VMEM((_KT, 16), jnp.float32),
            pltpu.SemaphoreType.DMA,
        ],
    )(_tapgather_body)


def _tapgather_body(xfT_hbm, tap_hbm, out_hbm, idx_v, rows_v, sem):
    cid = lax.axis_index("c")
    sid = lax.axis_index("s")
    w = cid * _NS + sid

    def body(j, carry):
        base = w * _TPW + j * _KT
        pltpu.sync_copy(tap_hbm.at[pl.ds(base, _KT)], idx_v)
        pltpu.async_copy(xfT_hbm.at[idx_v], rows_v, sem).wait()
        pltpu.sync_copy(rows_v, out_hbm.at[pl.ds(base, _KT)])
        return carry

    lax.fori_loop(0, _TNCH, body, 0)


def _edge_body(g_hbm, src_hbm, dst_hbm, zeros_hbm, out_hbm,
               src_v, dst_v, rows_v, acc, sem):
    cid = lax.axis_index("c")
    sid = lax.axis_index("s")
    w = cid * _NS + sid
    pltpu.sync_copy(zeros_hbm.at[pl.ds(sid * _RPT, _RPT)],
                    acc.at[pl.ds(sid * _RPT, _RPT)])
    plsc.subcore_barrier()

    def body(j, carry):
        base = w * _EPW + j * _KE
        pltpu.sync_copy(src_hbm.at[pl.ds(base, _KE)], src_v)
        pltpu.sync_copy(dst_hbm.at[pl.ds(base, _KE)], dst_v)
        pltpu.async_copy(g_hbm.at[src_v], rows_v, sem).wait()
        pltpu.sync_copy(rows_v, acc.at[dst_v], add=True)
        return carry

    lax.fori_loop(0, _NCH, body, 0)
    plsc.subcore_barrier()
    pltpu.sync_copy(acc.at[pl.ds(sid * _RPT, _RPT)],
                    out_hbm.at[cid, pl.ds(sid * _RPT, _RPT)])


def _bk_body(h_ref, w1_ref, degpT_ref, b1_ref, g_ref, sl_ref, dv_ref):
    h = h_ref[...]
    hp = lax.dot_general(h, w1_ref[...], (((1,), (1,)), ((), ())),
                         preferred_element_type=jnp.float32)
    dp = degpT_ref[...]
    deg = dp[:, 0:1] + dp[:, 1:2] + 1.0
    dinv = lax.rsqrt(deg)
    g_ref[...] = dinv * hp
    sl_ref[...] = dinv * dinv * hp + b1_ref[...]
    dv_ref[...] = jnp.broadcast_to(dinv, (N, HID_GCN))


def _ck_body(p_ref, sl_ref, dv_ref, w2_ref, b2_ref, g2_ref, sl2_ref):
    p = p_ref[...]
    dv = dv_ref[...]
    h1 = jnp.maximum(dv * (p[0] + p[1]) + sl_ref[...], 0.0)
    hp = lax.dot_general(h1, w2_ref[...], (((1,), (1,)), ((), ())),
                         preferred_element_type=jnp.float32)
    g2_ref[...] = dv * hp
    sl2_ref[...] = dv * dv * hp + b2_ref[...]


def _dk_body(p_ref, sl2_ref, dv_ref, batch_ref, lw_ref, lb_ref, out_ref):
    p = p_ref[...]
    h2 = dv_ref[...] * (p[0] + p[1]) + sl2_ref[...]
    bc = batch_ref[...]
    ids = lax.broadcasted_iota(jnp.int32, (1, B), 1)
    oneh = (bc == ids).astype(jnp.float32)
    sums = lax.dot_general(oneh, h2, (((0,), (0,)), ((), ())),
                           preferred_element_type=jnp.float32)
    cnts = lax.dot_general(oneh, jnp.ones((N, 1), jnp.float32),
                           (((0,), (0,)), ((), ())),
                           preferred_element_type=jnp.float32)
    pooled = sums / jnp.maximum(cnts, 1.0)
    logits = lax.dot_general(pooled, lw_ref[...], (((1,), (1,)), ((), ())),
                             preferred_element_type=jnp.float32) + lb_ref[...]
    z = logits - jnp.max(logits, axis=1, keepdims=True)
    e = jnp.exp(z)
    out_ref[...] = e / jnp.sum(e, axis=1, keepdims=True)


def _tc_call(body, out_shapes, *args):
    return pl.pallas_call(body, out_shape=out_shapes)(*args)


def kernel(x, edge_index, batch, conv_w, conv_b, dec_w, dec_b,
           W1, b1, W2, b2, lin_w, lin_b):
    f32 = jnp.float32
    xf = x.reshape(B, -1)
    xfT = jnp.pad(xf.T, ((0, 0), (0, 16 - B)))
    xgT = _tapgather_kernel_fn()(xfT, _TAPF)
    xg = xgT[:, :B].T.reshape(B, 4, _TPACK)
    convw = conv_w.reshape(NPG, CONV_HID, 4)
    convb = conv_b.reshape(NPG, 1, CONV_HID)
    dec_packed = jnp.concatenate(
        [jnp.pad(dec_w[r].reshape(HID_ENC, CONV_HID, _LS[r] - 1),
                 ((0, 0), (0, 0), (0, _PS[r] - (_LS[r] - 1))))
         for r in range(NPG)], axis=2)
    decb = jnp.stack(dec_b).reshape(NPG, 1, HID_ENC)

    enc = _encoder(xg, convw, convb, dec_packed, decb)
    h = enc.transpose(1, 0, 2).reshape(N, HID_ENC)

    ei = edge_index.astype(jnp.int32)
    src, dst = ei[0], ei[1]
    onesS = jnp.ones((_KE, _DW), f32)
    zerosS = jnp.zeros((N, _DW), f32)
    zerosNK = jnp.zeros((N, HID_GCN), f32)

    degp = _deg_kernel_fn()(dst, onesS, zerosS)
    degpT = degp[:, :, 0].T

    g1, sl1, dv = _tc_call(
        _bk_body,
        [jax.ShapeDtypeStruct((N, HID_GCN), f32)] * 3,
        h, W1, degpT, b1.reshape(1, HID_GCN))

    p1 = _edge_kernel_fn()(g1, src, dst, zerosNK)

    g2, sl2 = _tc_call(
        _ck_body,
        [jax.ShapeDtypeStruct((N, HID_GCN), f32)] * 2,
        p1, sl1, dv, W2, b2.reshape(1, HID_GCN))

    p2 = _edge_kernel_fn()(g2, src, dst, zerosNK)

    out = _tc_call(
        _dk_body,
        jax.ShapeDtypeStruct((B, NUM_CLASSES), f32),
        p2, sl2, dv, batch.astype(jnp.int32).reshape(N, 1),
        lin_w, lin_b.reshape(1, NUM_CLASSES))
    return out
